# baseline (speedup 1.0000x reference)
import jax
import jax.numpy as jnp
from jax.experimental import pallas as pl

N_IN = 64
N_OUT = 64
R = 8
D = 16
S = 25


def _body(w_ref, b_ref, out_ref):
    r = jnp.dot(w_ref[...], b_ref[0], preferred_element_type=jnp.float32)
    out_ref[...] = r.reshape(N_OUT, 1, N_IN, R * S)


def kernel(weights, basis):
    wf = weights.reshape(N_OUT * N_IN, D)
    bf = basis.reshape(D, R, R * S).transpose(1, 0, 2)
    out = pl.pallas_call(
        _body,
        grid=(R,),
        in_specs=[
            pl.BlockSpec((N_OUT * N_IN, D), lambda x: (0, 0)),
            pl.BlockSpec((1, D, R * S), lambda x: (x, 0, 0)),
        ],
        out_specs=pl.BlockSpec((N_OUT, 1, N_IN, R * S), lambda x: (0, x, 0, 0)),
        out_shape=jax.ShapeDtypeStruct((N_OUT, R, N_IN, R * S), jnp.float32),
    )(wf, bf)
    return out.reshape(N_OUT * R, N_IN * R, S)


# speedup vs baseline: 2.8874x; 2.8874x over previous
import jax
import jax.numpy as jnp
from jax.experimental import pallas as pl

N_IN = 64
N_OUT = 64
R = 8
D = 16
S = 25


def _body(w_ref, l_ref, selD_ref, sel64_ref, mask_ref, out_ref):
    w = w_ref[0]
    k1 = jnp.dot(selD_ref[...], w,
                 preferred_element_type=jnp.float32)
    k = jnp.dot(k1, sel64_ref[...],
                preferred_element_type=jnp.float32)
    k = k * mask_ref[...]
    r = jnp.dot(l_ref[...], k,
                preferred_element_type=jnp.float32)
    out_ref[...] = r.reshape(S, R, N_IN * R)


def kernel(weights, basis):
    wt = weights.reshape(N_OUT, N_IN, D).transpose(0, 2, 1)
    lmat = basis.transpose(3, 1, 0, 2).reshape(S * R, D * R)
    f32 = jnp.float32
    selD = (jnp.arange(D * R)[:, None] // R == jnp.arange(D)[None, :]).astype(f32)
    sel64 = (jnp.arange(N_IN)[:, None] == jnp.arange(N_IN * R)[None, :] // R).astype(f32)
    mask = (jnp.arange(D * R)[:, None] % R == jnp.arange(N_IN * R)[None, :] % R).astype(f32)
    p = pl.pallas_call(
        _body,
        grid=(N_OUT,),
        in_specs=[
            pl.BlockSpec((1, D, N_IN), lambda o: (o, 0, 0)),
            pl.BlockSpec((S * R, D * R), lambda o: (0, 0)),
            pl.BlockSpec((D * R, D), lambda o: (0, 0)),
            pl.BlockSpec((N_IN, N_IN * R), lambda o: (0, 0)),
            pl.BlockSpec((D * R, N_IN * R), lambda o: (0, 0)),
        ],
        out_specs=pl.BlockSpec((S, R, N_IN * R), lambda o: (0, o, 0)),
        out_shape=jax.ShapeDtypeStruct((S, N_OUT * R, N_IN * R), f32),
    )(wt, lmat, selD, sel64, mask)
    return p.transpose(1, 2, 0)
